# Initial kernel scaffold; baseline (speedup 1.0000x reference)
import jax
import jax.numpy as jnp
from jax.experimental import pallas as pl


def kernel(x, edge_index, W1, att_src1, att_dst1, b1, W2, att_src2, att_dst2, b2):
    raise NotImplementedError("write your pallas kernel here")


# speedup vs baseline: 20.0244x; 20.0244x over previous
import functools

import jax
import jax.numpy as jnp
from jax import lax
from jax.experimental import pallas as pl
from jax.experimental.pallas import tpu as pltpu
from jax.experimental.pallas import tpu_sc as plsc

_NC = 2
_NS = 16
_NW = _NC * _NS
_CH = 64
_AUG = 16


def _edge_pass(xw, a_src, a_dst, src, dst, n_real_edges):
    n_pad, c_aug = xw.shape
    e_pad = src.shape[0]
    ept = e_pad // _NW
    nch = ept // _CH
    rps = n_pad // _NS
    mesh = plsc.VectorSubcoreMesh(core_axis_name="c", subcore_axis_name="s")

    @functools.partial(
        pl.kernel,
        mesh=mesh,
        compiler_params=pltpu.CompilerParams(
            needs_layout_passes=False, use_tc_tiling_on_sc=False),
        out_type=jax.ShapeDtypeStruct((_NC, n_pad, c_aug), jnp.float32),
        scratch_types=[
            pltpu.VMEM((n_pad,), jnp.float32),
            pltpu.VMEM((n_pad,), jnp.float32),
            pltpu.VMEM((_CH,), jnp.int32),
            pltpu.VMEM((_CH,), jnp.int32),
            pltpu.VMEM((_CH, c_aug), jnp.float32),
            pltpu.VMEM((_CH,), jnp.float32),
            pltpu.VMEM_SHARED((n_pad, c_aug), jnp.float32),
            pltpu.SemaphoreType.DMA,
        ],
    )
    def k(xw_h, as_h, ad_h, src_h, dst_h, acc_o,
          as_v, ad_v, src_v, dst_v, rows_v, w_v, acc_sp, sem):
        cid = lax.axis_index("c")
        sid = lax.axis_index("s")
        wid = cid * _NS + sid
        lane = lax.iota(jnp.int32, 16)
        zl = jnp.zeros((16,), jnp.float32)

        pltpu.sync_copy(as_h, as_v)
        pltpu.sync_copy(ad_h, ad_v)

        def zrow(i, _):
            for kk in range(c_aug // 16):
                rows_v[i, pl.ds(16 * kk, 16)] = zl
            return 0
        lax.fori_loop(0, _CH, zrow, 0)
        rbase = sid * rps
        for j in range(rps // _CH):
            pltpu.sync_copy(rows_v, acc_sp.at[pl.ds(rbase + j * _CH, _CH)])
        plsc.subcore_barrier()

        def chunk(cix, _):
            base = wid * ept + cix * _CH
            pltpu.sync_copy(src_h.at[pl.ds(base, _CH)], src_v)
            pltpu.sync_copy(dst_h.at[pl.ds(base, _CH)], dst_v)
            cp = pltpu.async_copy(xw_h.at[src_v], rows_v, sem)
            for j in range(_CH // 16):
                s16 = src_v[pl.ds(16 * j, 16)]
                d16 = dst_v[pl.ds(16 * j, 16)]
                a = plsc.load_gather(as_v, [s16]) + plsc.load_gather(ad_v, [d16])
                a = jnp.maximum(a, 0.2 * a)
                w = jnp.exp(a)
                eid = base + 16 * j + lane
                w = jnp.where(eid < n_real_edges, w, 0.0)
                w_v[pl.ds(16 * j, 16)] = w
            cp.wait()
            def scale(i, _):
                ws = plsc.load_gather(w_v, [jnp.full((16,), i, jnp.int32)])
                for kk in range(c_aug // 16):
                    rows_v[i, pl.ds(16 * kk, 16)] = (
                        rows_v[i, pl.ds(16 * kk, 16)] * ws)
                return 0
            lax.fori_loop(0, _CH, scale, 0)
            pltpu.sync_copy(rows_v, acc_sp.at[dst_v], add=True)
            return 0
        lax.fori_loop(0, nch, chunk, 0)
        plsc.subcore_barrier()

        pltpu.sync_copy(acc_sp.at[pl.ds(rbase, rps)],
                        acc_o.at[cid, pl.ds(rbase, rps)])

    return k(xw, a_src, a_dst, src, dst)


def _dense1(x, W1a, atts, attd):
    n_pad, f_in = x.shape
    ca = W1a.shape[1]
    h = ca - _AUG
    br = 1024

    def body(x_r, w_r, s_r, d_r, xw_r, ws_r, as_r, ad_r):
        xw = jnp.dot(x_r[...], w_r[...], preferred_element_type=jnp.float32)
        col = lax.broadcasted_iota(jnp.int32, (br, ca), 1)
        xw = xw + jnp.where(col == h, 1.0, 0.0)
        xw_r[...] = xw
        a_s = jnp.sum(xw * s_r[...], axis=1, keepdims=True)
        a_d = jnp.sum(xw * d_r[...], axis=1, keepdims=True)
        as_r[...] = jnp.broadcast_to(a_s, (br, _AUG))
        ad_r[...] = jnp.broadcast_to(a_d, (br, _AUG))
        a = a_s + a_d
        a = jnp.maximum(a, 0.2 * a)
        ws_r[...] = jnp.broadcast_to(jnp.exp(a), (br, _AUG))

    return pl.pallas_call(
        body,
        grid=(n_pad // br,),
        in_specs=[
            pl.BlockSpec((br, f_in), lambda i: (i, 0)),
            pl.BlockSpec((f_in, ca), lambda i: (0, 0)),
            pl.BlockSpec((1, ca), lambda i: (0, 0)),
            pl.BlockSpec((1, ca), lambda i: (0, 0)),
        ],
        out_specs=[pl.BlockSpec((br, ca), lambda i: (i, 0))]
        + [pl.BlockSpec((br, _AUG), lambda i: (i, 0))] * 3,
        out_shape=[jax.ShapeDtypeStruct((n_pad, ca), jnp.float32)]
        + [jax.ShapeDtypeStruct((n_pad, _AUG), jnp.float32)] * 3,
    )(x, W1a, atts, attd)


def _dense2(acc_p, ws1, xw1, b1, W2a, atts2, attd2):
    n_pad, ca1 = xw1.shape
    h1 = ca1 - _AUG
    ca2 = W2a.shape[1]
    c2 = ca2 - _AUG
    br = 1024

    def body(ac_r, ws_r, xw_r, b_r, w2_r, s2_r, d2_r,
             xw2_r, ws2_r, as2_r, ad2_r):
        col1 = lax.broadcasted_iota(jnp.int32, (br, ca1), 1)
        sel1 = jnp.where(col1 == h1, 1.0, 0.0)
        num = ac_r[0] + ac_r[1] + ws_r[:, 0:1] * xw_r[...]
        den = jnp.sum(num * sel1, axis=1, keepdims=True) + 1e-16
        hid = jnp.maximum(num[:, :h1] / den + b_r[...], 0.0)
        xw2 = jnp.dot(hid, w2_r[...], preferred_element_type=jnp.float32)
        col2 = lax.broadcasted_iota(jnp.int32, (br, ca2), 1)
        xw2 = xw2 + jnp.where(col2 == c2, 1.0, 0.0)
        xw2_r[...] = xw2
        a_s = jnp.sum(xw2 * s2_r[...], axis=1, keepdims=True)
        a_d = jnp.sum(xw2 * d2_r[...], axis=1, keepdims=True)
        as2_r[...] = jnp.broadcast_to(a_s, (br, _AUG))
        ad2_r[...] = jnp.broadcast_to(a_d, (br, _AUG))
        a = a_s + a_d
        a = jnp.maximum(a, 0.2 * a)
        ws2_r[...] = jnp.broadcast_to(jnp.exp(a), (br, _AUG))

    return pl.pallas_call(
        body,
        grid=(n_pad // br,),
        in_specs=[
            pl.BlockSpec((2, br, ca1), lambda i: (0, i, 0)),
            pl.BlockSpec((br, _AUG), lambda i: (i, 0)),
            pl.BlockSpec((br, ca1), lambda i: (i, 0)),
            pl.BlockSpec((1, h1), lambda i: (0, 0)),
            pl.BlockSpec((h1, ca2), lambda i: (0, 0)),
            pl.BlockSpec((1, ca2), lambda i: (0, 0)),
            pl.BlockSpec((1, ca2), lambda i: (0, 0)),
        ],
        out_specs=[pl.BlockSpec((br, ca2), lambda i: (i, 0))]
        + [pl.BlockSpec((br, _AUG), lambda i: (i, 0))] * 3,
        out_shape=[jax.ShapeDtypeStruct((n_pad, ca2), jnp.float32)]
        + [jax.ShapeDtypeStruct((n_pad, _AUG), jnp.float32)] * 3,
    )(acc_p, ws1, xw1, b1, W2a, atts2, attd2)


def _dense3(acc_p, ws2, xw2, b2):
    n_pad, ca2 = xw2.shape
    c2 = ca2 - _AUG
    br = 1024

    def body(ac_r, ws_r, xw_r, b_r, o_r):
        col2 = lax.broadcasted_iota(jnp.int32, (br, ca2), 1)
        sel2 = jnp.where(col2 == c2, 1.0, 0.0)
        num = ac_r[0] + ac_r[1] + ws_r[:, 0:1] * xw_r[...]
        den = jnp.sum(num * sel2, axis=1, keepdims=True) + 1e-16
        o_r[...] = num[:, :c2] / den + b_r[...]

    return pl.pallas_call(
        body,
        grid=(n_pad // br,),
        in_specs=[
            pl.BlockSpec((2, br, ca2), lambda i: (0, i, 0)),
            pl.BlockSpec((br, _AUG), lambda i: (i, 0)),
            pl.BlockSpec((br, ca2), lambda i: (i, 0)),
            pl.BlockSpec((1, c2), lambda i: (0, 0)),
        ],
        out_specs=pl.BlockSpec((br, c2), lambda i: (i, 0)),
        out_shape=jax.ShapeDtypeStruct((n_pad, c2), jnp.float32),
    )(acc_p, ws2, xw2, b2)


def kernel(x, edge_index, W1, att_src1, att_dst1, b1, W2, att_src2, att_dst2, b2):
    n, f_in = x.shape
    e = edge_index.shape[1]
    n_pad = ((n + 2047) // 2048) * 2048
    e_pad = ((e + _NW * _CH - 1) // (_NW * _CH)) * (_NW * _CH)

    xp = jnp.pad(x, ((0, n_pad - n), (0, 0)))
    src = jnp.pad(edge_index[0], (0, e_pad - e))
    dst = jnp.pad(edge_index[1], (0, e_pad - e))

    W1a = jnp.pad(W1, ((0, 0), (0, _AUG)))
    atts1 = jnp.pad(att_src1.reshape(1, -1), ((0, 0), (0, _AUG)))
    attd1 = jnp.pad(att_dst1.reshape(1, -1), ((0, 0), (0, _AUG)))
    W2a = jnp.pad(W2, ((0, 0), (0, _AUG)))
    atts2 = jnp.pad(att_src2.reshape(1, -1), ((0, 0), (0, _AUG)))
    attd2 = jnp.pad(att_dst2.reshape(1, -1), ((0, 0), (0, _AUG)))

    xw1, ws1, as1, ad1 = _dense1(xp, W1a, atts1, attd1)
    acc1 = _edge_pass(xw1, as1[:, 0], ad1[:, 0], src, dst, e)
    xw2, ws2, as2, ad2 = _dense2(acc1, ws1, xw1, b1.reshape(1, -1),
                                 W2a, atts2, attd2)
    acc2 = _edge_pass(xw2, as2[:, 0], ad2[:, 0], src, dst, e)
    out = _dense3(acc2, ws2, xw2, b2.reshape(1, -1))
    return out[:n]
